# baseline (speedup 1.0000x reference)
import functools

import jax
import jax.numpy as jnp
from jax import lax
from jax.experimental import pallas as pl
from jax.experimental.pallas import tpu as pltpu


def _conv3x3(x3e_ref, w_ref, c, marg, npad, wp):
    acc = None
    for dy in range(3):
        a0 = marg + (dy - 1) * wp
        d = jnp.dot(x3e_ref[a0:a0 + npad, :],
                    w_ref[dy * 3 * c:(dy + 1) * 3 * c, :],
                    preferred_element_type=jnp.float32)
        acc = d if acc is None else acc + d
    return acc


def _build_wide(x3e_ref, ntot, c):
    x3e_ref[1:ntot, 0:c] = x3e_ref[0:ntot - 1, c:2 * c]
    x3e_ref[0:1, 0:c] = jnp.zeros((1, c), x3e_ref.dtype)
    x3e_ref[0:ntot - 1, 2 * c:3 * c] = x3e_ref[1:ntot, c:2 * c]
    x3e_ref[ntot - 1:ntot, 2 * c:3 * c] = jnp.zeros((1, c), x3e_ref.dtype)


def _pass_a(x_ref, mask_ref, w1_ref, ws_ref, y1_ref, skip_ref, stats_ref,
            x3e_ref, *, nb, cin, cout, h, w, wp, marg):
    npad = h * wp
    ntot = 2 * marg + npad
    for i in range(nb):
        xb = x_ref[i].astype(jnp.bfloat16)
        skip_ref[i, :, :] = jnp.dot(
            xb, ws_ref[...], preferred_element_type=jnp.float32
        ).astype(skip_ref.dtype)
        x3e_ref[:, cin:2 * cin] = jnp.zeros((ntot, cin), x3e_ref.dtype)
        for y in range(h):
            d0 = marg + y * wp
            x3e_ref[d0:d0 + w, cin:2 * cin] = xb[y * w:(y + 1) * w, :]
        _build_wide(x3e_ref, ntot, cin)
        yv = _conv3x3(x3e_ref, w1_ref, cin, marg, npad, wp)
        hm = yv * mask_ref[...]
        stats_ref[i, 0:1, :] = jnp.sum(hm, axis=0, keepdims=True)
        stats_ref[i, 1:2, :] = jnp.sum(hm * hm, axis=0, keepdims=True)
        yb = yv.astype(y1_ref.dtype)
        for y in range(h):
            y1_ref[i, y * w:(y + 1) * w, :] = yb[y * wp:y * wp + w, :]


def _pass_b(y1_ref, mask_ref, sc1_ref, sh1_ref, w2_ref, h2_ref, stats_ref,
            x3e_ref, *, nb, cout, h, w, wp, marg):
    npad = h * wp
    ntot = 2 * marg + npad
    for i in range(nb):
        h1 = jnp.maximum(sc1_ref[...] * y1_ref[i].astype(jnp.float32)
                         + sh1_ref[...], 0.0).astype(x3e_ref.dtype)
        x3e_ref[:, cout:2 * cout] = jnp.zeros((ntot, cout), x3e_ref.dtype)
        for y in range(h):
            d0 = marg + y * wp
            x3e_ref[d0:d0 + w, cout:2 * cout] = h1[y * w:(y + 1) * w, :]
        _build_wide(x3e_ref, ntot, cout)
        yv = _conv3x3(x3e_ref, w2_ref, cout, marg, npad, wp)
        hm = yv * mask_ref[...]
        stats_ref[i, 0:1, :] = jnp.sum(hm, axis=0, keepdims=True)
        stats_ref[i, 1:2, :] = jnp.sum(hm * hm, axis=0, keepdims=True)
        yb = yv.astype(h2_ref.dtype)
        for y in range(h):
            s0 = y * wp
            h2_ref[i, y * w:(y + 1) * w, :] = yb[s0:s0 + w, :]


def _pass_c(h2_ref, skip_ref, sc2_ref, sh2_ref, out_ref):
    out_ref[...] = jnp.maximum(
        sc2_ref[...] * h2_ref[...].astype(jnp.float32)
        + sh2_ref[...] + skip_ref[...].astype(jnp.float32), 0.0
    ).astype(out_ref.dtype)


def kernel(x, w1, b1, g1, be1, w2, b2, g2, be2, ws, bs, *, eps=1e-5):
    N, Cin, H, W = x.shape
    Cout = w1.shape[-1]
    Wp = -(-(W + 2) // 16) * 16
    Npad = H * Wp
    HW = H * W
    marg = Wp
    f32 = jnp.float32
    bf16 = jnp.bfloat16
    dtype = x.dtype

    xt = x.reshape(N, Cin, HW).transpose(0, 2, 1)

    q = jnp.arange(Npad)
    mask = (q % Wp < W).astype(f32).reshape(Npad, 1)

    w1mat = w1.reshape(9 * Cin, Cout).astype(bf16)
    w2mat = w2.reshape(9 * Cout, Cout).astype(bf16)
    wsmat = ws.astype(bf16)

    cparams = pltpu.CompilerParams(
        dimension_semantics=("parallel",),
        vmem_limit_bytes=56 * 1024 * 1024)

    def full(shape):
        return pl.BlockSpec(shape, lambda n: (0,) * len(shape))

    def per_n(r, c):
        return pl.BlockSpec((None, r, c), lambda n: (n, 0, 0))

    nb = 4 if N % 4 == 0 else (2 if N % 2 == 0 else 1)

    def per_nb(r, c):
        return pl.BlockSpec((nb, r, c), lambda n: (n, 0, 0))

    ntot = 2 * marg + Npad

    y1, skip, stats1 = pl.pallas_call(
        functools.partial(_pass_a, nb=nb, cin=Cin, cout=Cout, h=H, w=W,
                          wp=Wp, marg=marg),
        grid=(N // nb,),
        in_specs=[per_nb(HW, Cin), full((Npad, 1)),
                  full((9 * Cin, Cout)), full((Cin, Cout))],
        out_specs=[per_nb(HW, Cout), per_nb(HW, Cout), per_nb(2, Cout)],
        out_shape=[jax.ShapeDtypeStruct((N, HW, Cout), bf16),
                   jax.ShapeDtypeStruct((N, HW, Cout), bf16),
                   jax.ShapeDtypeStruct((N, 2, Cout), f32)],
        scratch_shapes=[pltpu.VMEM((ntot, 3 * Cin), bf16)],
        compiler_params=cparams,
    )(xt, mask, w1mat, wsmat)

    cnt = float(N * H * W)
    tot1 = jnp.sum(stats1, axis=0)
    mean1 = tot1[0] / cnt
    var1 = jnp.maximum(tot1[1] / cnt - mean1 * mean1, 0.0)
    sc1 = (g1.reshape(Cout) * lax.rsqrt(var1 + eps)).reshape(1, Cout)
    sh1 = (be1.reshape(Cout) - sc1[0] * mean1).reshape(1, Cout)

    h2, stats2 = pl.pallas_call(
        functools.partial(_pass_b, nb=nb, cout=Cout, h=H, w=W,
                          wp=Wp, marg=marg),
        grid=(N // nb,),
        in_specs=[per_nb(HW, Cout), full((Npad, 1)), full((1, Cout)),
                  full((1, Cout)), full((9 * Cout, Cout))],
        out_specs=[per_nb(HW, Cout), per_nb(2, Cout)],
        out_shape=[jax.ShapeDtypeStruct((N, HW, Cout), bf16),
                   jax.ShapeDtypeStruct((N, 2, Cout), f32)],
        scratch_shapes=[pltpu.VMEM((ntot, 3 * Cout), bf16)],
        compiler_params=cparams,
    )(y1, mask, sc1.astype(f32), sh1.astype(f32), w2mat)

    tot2 = jnp.sum(stats2, axis=0)
    mean2 = tot2[0] / cnt
    var2 = jnp.maximum(tot2[1] / cnt - mean2 * mean2, 0.0)
    sc2 = (g2.reshape(Cout) * lax.rsqrt(var2 + eps)).reshape(1, Cout)
    sh2 = (be2.reshape(Cout) - sc2[0] * mean2 + bs.reshape(Cout)).reshape(1, Cout)

    out = pl.pallas_call(
        _pass_c,
        grid=(N,),
        in_specs=[per_n(HW, Cout), per_n(HW, Cout),
                  full((1, Cout)), full((1, Cout))],
        out_specs=per_n(HW, Cout),
        out_shape=jax.ShapeDtypeStruct((N, HW, Cout), dtype),
        compiler_params=cparams,
    )(h2, skip, sc2.astype(f32), sh2.astype(f32))

    return out.transpose(0, 2, 1).reshape(N, Cout, H, W)


# speedup vs baseline: 1.0281x; 1.0281x over previous
import functools

import jax
import jax.numpy as jnp
from jax import lax
from jax.experimental import pallas as pl
from jax.experimental.pallas import tpu as pltpu


def _conv3x3(x3e_ref, w_ref, c, marg, npad, wp):
    acc = None
    for dy in range(3):
        a0 = marg + (dy - 1) * wp
        d = jnp.dot(x3e_ref[a0:a0 + npad, :],
                    w_ref[dy * 3 * c:(dy + 1) * 3 * c, :],
                    preferred_element_type=jnp.float32)
        acc = d if acc is None else acc + d
    return acc


def _build_wide(x3e_ref, ntot, c):
    x3e_ref[1:ntot, 0:c] = x3e_ref[0:ntot - 1, c:2 * c]
    x3e_ref[0:1, 0:c] = jnp.zeros((1, c), x3e_ref.dtype)
    x3e_ref[0:ntot - 1, 2 * c:3 * c] = x3e_ref[1:ntot, c:2 * c]
    x3e_ref[ntot - 1:ntot, 2 * c:3 * c] = jnp.zeros((1, c), x3e_ref.dtype)


def _pass_a(x_ref, mask_ref, w1_ref, ws_ref, y1_ref, skip_ref, stats_ref,
            x3e_ref, *, nb, cin, cout, h, w, wp, marg):
    npad = h * wp
    ntot = 2 * marg + npad
    for i in range(nb):
        xb = x_ref[i].astype(jnp.bfloat16)
        skip_ref[i, :, :] = jnp.dot(
            xb, ws_ref[...], preferred_element_type=jnp.float32
        ).astype(skip_ref.dtype)
        x3e_ref[:, cin:2 * cin] = jnp.zeros(
            (x3e_ref.shape[0], cin), x3e_ref.dtype)
        for y in range(h):
            d0 = marg + y * wp
            x3e_ref[d0:d0 + w, cin:2 * cin] = xb[y * w:(y + 1) * w, :]
        _build_wide(x3e_ref, ntot, cin)
        yv = _conv3x3(x3e_ref, w1_ref, cin, marg, npad, wp)
        y1_ref[i, :, :] = yv.astype(y1_ref.dtype)
        hm = yv * mask_ref[...]
        stats_ref[i, 0:1, :] = jnp.sum(hm, axis=0, keepdims=True)
        stats_ref[i, 1:2, :] = jnp.sum(hm * hm, axis=0, keepdims=True)


def _pass_b(y1_ref, mask_ref, sc1_ref, sh1_ref, w2_ref, h2_ref, stats_ref,
            x3e_ref, *, nb, cout, h, w, wp, marg):
    npad = h * wp
    ntot = 2 * marg + npad
    for i in range(nb):
        h1 = jnp.maximum(sc1_ref[...] * y1_ref[i].astype(jnp.float32)
                         + sh1_ref[...], 0.0) * mask_ref[...]
        x3e_ref[0:marg, cout:2 * cout] = jnp.zeros((marg, cout), x3e_ref.dtype)
        x3e_ref[marg:marg + npad, cout:2 * cout] = h1.astype(x3e_ref.dtype)
        x3e_ref[marg + npad:, cout:2 * cout] = jnp.zeros(
            (marg, cout), x3e_ref.dtype)
        _build_wide(x3e_ref, ntot, cout)
        yv = _conv3x3(x3e_ref, w2_ref, cout, marg, npad, wp)
        hm = yv * mask_ref[...]
        stats_ref[i, 0:1, :] = jnp.sum(hm, axis=0, keepdims=True)
        stats_ref[i, 1:2, :] = jnp.sum(hm * hm, axis=0, keepdims=True)
        yb = yv.astype(h2_ref.dtype)
        for y in range(h):
            s0 = y * wp
            h2_ref[i, y * w:(y + 1) * w, :] = yb[s0:s0 + w, :]


def _pass_c(h2_ref, skip_ref, sc2_ref, sh2_ref, out_ref):
    out_ref[...] = jnp.maximum(
        sc2_ref[...] * h2_ref[...].astype(jnp.float32)
        + sh2_ref[...] + skip_ref[...].astype(jnp.float32), 0.0
    ).astype(out_ref.dtype)


def kernel(x, w1, b1, g1, be1, w2, b2, g2, be2, ws, bs, *, eps=1e-5):
    N, Cin, H, W = x.shape
    Cout = w1.shape[-1]
    Wp = -(-(W + 2) // 16) * 16
    Npad = H * Wp
    HW = H * W
    marg = Wp
    f32 = jnp.float32
    bf16 = jnp.bfloat16
    dtype = x.dtype

    xt = x.reshape(N, Cin, HW).transpose(0, 2, 1)

    q = jnp.arange(Npad)
    mask = (q % Wp < W).astype(f32).reshape(Npad, 1)

    w1mat = w1.reshape(9 * Cin, Cout).astype(bf16)
    w2mat = w2.reshape(9 * Cout, Cout).astype(bf16)
    wsmat = ws.astype(bf16)

    cparams = pltpu.CompilerParams(
        dimension_semantics=("parallel",),
        vmem_limit_bytes=56 * 1024 * 1024)

    def full(shape):
        return pl.BlockSpec(shape, lambda n: (0,) * len(shape))

    def per_n(r, c):
        return pl.BlockSpec((None, r, c), lambda n: (n, 0, 0))

    nb = 4 if N % 4 == 0 else (2 if N % 2 == 0 else 1)

    def per_nb(r, c):
        return pl.BlockSpec((nb, r, c), lambda n: (n, 0, 0))

    ntot = 2 * marg + Npad

    y1, skip, stats1 = pl.pallas_call(
        functools.partial(_pass_a, nb=nb, cin=Cin, cout=Cout, h=H, w=W,
                          wp=Wp, marg=marg),
        grid=(N // nb,),
        in_specs=[per_nb(HW, Cin), full((Npad, 1)),
                  full((9 * Cin, Cout)), full((Cin, Cout))],
        out_specs=[per_nb(Npad, Cout), per_nb(HW, Cout), per_nb(2, Cout)],
        out_shape=[jax.ShapeDtypeStruct((N, Npad, Cout), bf16),
                   jax.ShapeDtypeStruct((N, HW, Cout), bf16),
                   jax.ShapeDtypeStruct((N, 2, Cout), f32)],
        scratch_shapes=[pltpu.VMEM((ntot, 3 * Cin), bf16)],
        compiler_params=cparams,
    )(xt, mask, w1mat, wsmat)

    cnt = float(N * H * W)
    tot1 = jnp.sum(stats1, axis=0)
    mean1 = tot1[0] / cnt
    var1 = jnp.maximum(tot1[1] / cnt - mean1 * mean1, 0.0)
    sc1 = (g1.reshape(Cout) * lax.rsqrt(var1 + eps)).reshape(1, Cout)
    sh1 = (be1.reshape(Cout) - sc1[0] * mean1).reshape(1, Cout)

    h2, stats2 = pl.pallas_call(
        functools.partial(_pass_b, nb=nb, cout=Cout, h=H, w=W,
                          wp=Wp, marg=marg),
        grid=(N // nb,),
        in_specs=[per_nb(Npad, Cout), full((Npad, 1)), full((1, Cout)),
                  full((1, Cout)), full((9 * Cout, Cout))],
        out_specs=[per_nb(HW, Cout), per_nb(2, Cout)],
        out_shape=[jax.ShapeDtypeStruct((N, HW, Cout), bf16),
                   jax.ShapeDtypeStruct((N, 2, Cout), f32)],
        scratch_shapes=[pltpu.VMEM((ntot, 3 * Cout), bf16)],
        compiler_params=cparams,
    )(y1, mask, sc1.astype(f32), sh1.astype(f32), w2mat)

    tot2 = jnp.sum(stats2, axis=0)
    mean2 = tot2[0] / cnt
    var2 = jnp.maximum(tot2[1] / cnt - mean2 * mean2, 0.0)
    sc2 = (g2.reshape(Cout) * lax.rsqrt(var2 + eps)).reshape(1, Cout)
    sh2 = (be2.reshape(Cout) - sc2[0] * mean2 + bs.reshape(Cout)).reshape(1, Cout)

    out = pl.pallas_call(
        _pass_c,
        grid=(N,),
        in_specs=[per_n(HW, Cout), per_n(HW, Cout),
                  full((1, Cout)), full((1, Cout))],
        out_specs=per_n(HW, Cout),
        out_shape=jax.ShapeDtypeStruct((N, HW, Cout), dtype),
        compiler_params=cparams,
    )(h2, skip, sc2.astype(f32), sh2.astype(f32))

    return out.transpose(0, 2, 1).reshape(N, Cout, H, W)
